# Initial kernel scaffold; baseline (speedup 1.0000x reference)
import jax
import jax.numpy as jnp
from jax.experimental import pallas as pl


def kernel(batch, x, edge_index, edge_attr, params):
    raise NotImplementedError("write your pallas kernel here")


# speedup vs baseline: 7.0514x; 7.0514x over previous
import functools

import jax
import jax.numpy as jnp
from jax import lax
from jax.experimental import pallas as pl
from jax.experimental.pallas import tpu as pltpu
from jax.experimental.pallas import tpu_sc as plsc

N = 10000
E = 320000
D = 128
L = 5

NC = 2
NS = 16
NW = NC * NS
EPT = E // NW
CHUNK = 40
NCH = EPT // CHUNK
NBUF = 2
ROUNDS = NCH // NBUF
ROWS_PT = 624
TAIL = N - NS * ROWS_PT
GROUPS = D // 16


def _sc_body(x_hbm, src_hbm, dst_hbm, ea_hbm, out_hbm,
             srcs_v, dsts_v, rows_v, ea_v, msg_v, acc_sh,
             sem_in, sem_sc):
    c = lax.axis_index("c")
    s = lax.axis_index("s")
    wid = c * NS + s
    ebase = wid * EPT

    pltpu.sync_copy(src_hbm.at[wid], srcs_v)
    pltpu.sync_copy(dst_hbm.at[wid], dsts_v)

    def _fire(b, cidx):
        pltpu.async_copy(x_hbm.at[srcs_v.at[cidx]], rows_v.at[b], sem_in.at[b])
        pltpu.async_copy(ea_hbm.at[pl.ds(ebase + cidx * CHUNK, CHUNK), :],
                         ea_v.at[b], sem_in.at[b])

    def _wait_in(b, cidx):
        pltpu.make_async_copy(x_hbm.at[srcs_v.at[cidx]], rows_v.at[b],
                              sem_in.at[b]).wait()
        pltpu.make_async_copy(ea_hbm.at[pl.ds(ebase + cidx * CHUNK, CHUNK), :],
                              ea_v.at[b], sem_in.at[b]).wait()

    def _wait_sc(b, cidx):
        pltpu.make_async_copy(msg_v.at[b], acc_sh.at[dsts_v.at[cidx]],
                              sem_sc.at[b]).wait()

    for b in range(NBUF):
        _fire(b, b)

    zero = jnp.zeros((16,), jnp.float32)

    def _zfill(i, carry):
        for g in range(GROUPS):
            msg_v[0, i, pl.ds(g * 16, 16)] = zero
        return carry

    lax.fori_loop(0, CHUNK, _zfill, 0)
    for k in range(ROWS_PT // CHUNK):
        pltpu.sync_copy(msg_v.at[0],
                        acc_sh.at[pl.ds(s * ROWS_PT + k * CHUNK, CHUNK), :])
    pltpu.sync_copy(msg_v.at[0, pl.ds(0, ROWS_PT % CHUNK), :],
                    acc_sh.at[pl.ds(s * ROWS_PT + (ROWS_PT // CHUNK) * CHUNK,
                                    ROWS_PT % CHUNK), :])

    @pl.when(s == 0)
    def _():
        pltpu.sync_copy(msg_v.at[0, pl.ds(0, TAIL), :],
                        acc_sh.at[pl.ds(NS * ROWS_PT, TAIL), :])

    plsc.subcore_barrier()

    def _round(r, carry):
        for b in range(NBUF):
            cidx = r * NBUF + b
            _wait_in(b, cidx)

            rowsb, eab, msgb = rows_v.at[b], ea_v.at[b], msg_v.at[b]

            def _row(i, cc):
                for g in range(GROUPS):
                    sl = pl.ds(g * 16, 16)
                    msgb[i, sl] = jnp.maximum(rowsb[i, sl] + eab[i, sl], 0.0)
                return cc

            lax.fori_loop(0, CHUNK, _row, 0)

            @pl.when(r < ROUNDS - 1)
            def _():
                _fire(b, cidx + NBUF)

            pltpu.sync_copy(msg_v.at[b], acc_sh.at[dsts_v.at[cidx]], add=True)
        return carry

    lax.fori_loop(0, ROUNDS, _round, 0)

    plsc.subcore_barrier()
    plsc.subcore_barrier()

    pltpu.sync_copy(acc_sh.at[pl.ds(s * ROWS_PT, ROWS_PT), :],
                    out_hbm.at[c, pl.ds(s * ROWS_PT, ROWS_PT), :])

    @pl.when(s == 0)
    def _():
        pltpu.sync_copy(acc_sh.at[pl.ds(NS * ROWS_PT, TAIL), :],
                        out_hbm.at[c, pl.ds(NS * ROWS_PT, TAIL), :])


@functools.cache
def _make_sc_mp():
  return pl.kernel(
    _sc_body,
    out_type=jax.ShapeDtypeStruct((NC, N, D), jnp.float32),
    mesh=plsc.VectorSubcoreMesh(core_axis_name="c", subcore_axis_name="s"),
    compiler_params=pltpu.CompilerParams(use_tc_tiling_on_sc=False),
    scratch_types=[
        pltpu.VMEM((NCH, CHUNK), jnp.int32),
        pltpu.VMEM((NCH, CHUNK), jnp.int32),
        pltpu.VMEM((NBUF, CHUNK, D), jnp.float32),
        pltpu.VMEM((NBUF, CHUNK, D), jnp.float32),
        pltpu.VMEM((NBUF, CHUNK, D), jnp.float32),
        pltpu.VMEM_SHARED((N, D), jnp.float32),
        pltpu.SemaphoreType.DMA((NBUF,)),
        pltpu.SemaphoreType.DMA((NBUF,)),
    ],
  )


def _colmean(x):
    n = x.shape[0]
    carry = None
    while n > 1:
        if n % 2 == 1:
            row = x[n - 1:n]
            carry = row if carry is None else carry + row
            n -= 1
        h = n // 2
        x = x[:h] + x[h:n]
        n = h
    if carry is not None:
        x = x + carry
    return x * (1.0 / N)


def _bn(h, g, be):
    m = _colmean(h)
    v = _colmean((h - m) ** 2) + 1e-5
    r = lax.rsqrt(v)
    r = r * (1.5 - 0.5 * v * r * r)
    return g * ((h - m) * r) + be


def _mlp_body(last, h_ref, acc_ref, w1_ref, b1_ref, g1_ref, be1_ref,
              w2_ref, b2_ref, g3_ref, be3_ref, o_ref):
    z = h_ref[...] + acc_ref[0] + acc_ref[1]
    h1 = jnp.dot(z, w1_ref[...], preferred_element_type=jnp.float32) + b1_ref[...]
    h1 = jnp.maximum(_bn(h1, g1_ref[...], be1_ref[...]), 0.0)
    h2 = jnp.dot(h1, w2_ref[...], preferred_element_type=jnp.float32) + b2_ref[...]
    o = _bn(h2, g3_ref[...], be3_ref[...])
    if not last:
        o = jnp.maximum(o, 0.0)
    o_ref[...] = o


def _mlp(last):
    return pl.pallas_call(
        functools.partial(_mlp_body, last),
        out_shape=jax.ShapeDtypeStruct((N, D), jnp.float32),
    )


def kernel(batch, x, edge_index, edge_attr, params):
    src_r = edge_index[0].astype(jnp.int32).reshape(NW, NCH, CHUNK)
    dst_r = edge_index[1].astype(jnp.int32).reshape(NW, NCH, CHUNK)
    h = x
    xs = []
    for i in range(L):
        p = params[i]
        acc2 = _make_sc_mp()(h, src_r, dst_r, edge_attr)
        h = _mlp(i == L - 1)(
            h, acc2,
            p['W1'], p['b1'].reshape(1, 2 * D),
            p['g1'].reshape(1, 2 * D), p['be1'].reshape(1, 2 * D),
            p['W2'], p['b2'].reshape(1, D),
            p['g3'].reshape(1, D), p['be3'].reshape(1, D),
        )
        xs.append(h)
    return (h, tuple(xs))
